# baseline (speedup 1.0000x reference)
import functools

import jax
import jax.numpy as jnp
from jax import lax
from jax.experimental import pallas as pl
from jax.experimental.pallas import tpu as pltpu
from jax.experimental.pallas import tpu_sc as plsc

_N = 100000
_OBS_D = 17
_ACT_D = 6
_NW = 32
_ROWS = 3136
_CHUNK = 448
_NCHUNK = _ROWS // _CHUNK

_TC_BLOCK = 2000


def _sc_obs_body(obs_hbm, obs_out, obs_v, sem_in, sem_out):
    c = lax.axis_index("c")
    s = lax.axis_index("s")
    wid = s * 2 + c
    base = jnp.minimum(wid * _ROWS, _N - _ROWS)

    outs = [None] * _NCHUNK
    for i in range(_NCHUNK):
        b = i % 2
        if i >= 2:
            outs[i - 2].wait()
        lo = base + i * _CHUNK
        pltpu.async_copy(
            obs_hbm.at[pl.ds(lo, _CHUNK), :], obs_v.at[b], sem_in).wait()
        outs[i] = pltpu.async_copy(
            obs_v.at[b], obs_out.at[pl.ds(lo, _CHUNK), :], sem_out)
    outs[_NCHUNK - 2].wait()
    outs[_NCHUNK - 1].wait()


def _tc_act_body(act_ref, act_out):
    act_out[...] = act_ref[...]


def kernel(obs_table, act_table):
    sc_copy = functools.partial(
        pl.kernel,
        out_type=jax.ShapeDtypeStruct((_N, _OBS_D), jnp.float32),
        mesh=plsc.VectorSubcoreMesh(core_axis_name="c", subcore_axis_name="s"),
        scratch_types=[
            pltpu.VMEM((2, _CHUNK, _OBS_D), jnp.float32),
            pltpu.SemaphoreType.DMA,
            pltpu.SemaphoreType.DMA,
        ],
    )(_sc_obs_body)
    obs_o = sc_copy(obs_table)

    act_o = pl.pallas_call(
        _tc_act_body,
        grid=(_N // _TC_BLOCK,),
        in_specs=[pl.BlockSpec((_TC_BLOCK, _ACT_D), lambda i: (i, 0))],
        out_specs=pl.BlockSpec((_TC_BLOCK, _ACT_D), lambda i: (i, 0)),
        out_shape=jax.ShapeDtypeStruct((_N, _ACT_D), jnp.float32),
    )(act_table)

    return (obs_o, act_o)


# speedup vs baseline: 1.0279x; 1.0279x over previous
import functools

import jax
import jax.numpy as jnp
from jax import lax
from jax.experimental import pallas as pl
from jax.experimental.pallas import tpu as pltpu
from jax.experimental.pallas import tpu_sc as plsc

_N = 100000
_OBS_D = 17
_ACT_D = 6
_NW = 32
_ROWS = 3136
_CHUNK = 112
_NCHUNK = _ROWS // _CHUNK
_NBUF = 3


def _sc_copy_body(obs_hbm, act_hbm, obs_out, act_out,
                  obs_v, act_v, sem_in, sem_out):
    c = lax.axis_index("c")
    s = lax.axis_index("s")
    wid = s * 2 + c
    base = jnp.minimum(wid * _ROWS, _N - _ROWS)

    def start_in(i):
        b = i % _NBUF
        lo = base + i * _CHUNK
        return (
            pltpu.async_copy(
                obs_hbm.at[pl.ds(lo, _CHUNK), :], obs_v.at[b], sem_in),
            pltpu.async_copy(
                act_hbm.at[pl.ds(lo, _CHUNK), :], act_v.at[b], sem_in),
        )

    def start_out(i):
        b = i % _NBUF
        lo = base + i * _CHUNK
        return (
            pltpu.async_copy(
                obs_v.at[b], obs_out.at[pl.ds(lo, _CHUNK), :], sem_out),
            pltpu.async_copy(
                act_v.at[b], act_out.at[pl.ds(lo, _CHUNK), :], sem_out),
        )

    ins = [None] * _NCHUNK
    outs = [None] * _NCHUNK
    ins[0] = start_in(0)
    for i in range(_NCHUNK):
        nxt = i + 1
        if nxt < _NCHUNK:
            if nxt >= _NBUF:
                for cp in outs[nxt - _NBUF]:
                    cp.wait()
            ins[nxt] = start_in(nxt)
        for cp in ins[i]:
            cp.wait()
        outs[i] = start_out(i)
    for i in range(_NCHUNK - _NBUF, _NCHUNK):
        for cp in outs[i]:
            cp.wait()


def kernel(obs_table, act_table):
    k = functools.partial(
        pl.kernel,
        out_type=(
            jax.ShapeDtypeStruct((_N, _OBS_D), jnp.float32),
            jax.ShapeDtypeStruct((_N, _ACT_D), jnp.float32),
        ),
        mesh=plsc.VectorSubcoreMesh(core_axis_name="c", subcore_axis_name="s"),
        scratch_types=[
            pltpu.VMEM((_NBUF, _CHUNK, _OBS_D), jnp.float32),
            pltpu.VMEM((_NBUF, _CHUNK, _ACT_D), jnp.float32),
            pltpu.SemaphoreType.DMA,
            pltpu.SemaphoreType.DMA,
        ],
    )(_sc_copy_body)
    return k(obs_table, act_table)


# speedup vs baseline: 1.1258x; 1.0953x over previous
import jax
import jax.numpy as jnp
from jax.experimental import pallas as pl
from jax.experimental.pallas import tpu as pltpu


def _copy_body(obs_ref, act_ref, obs_out, act_out):
    obs_out[...] = obs_ref[...]
    act_out[...] = act_ref[...]


def kernel(obs_table, act_table):
    n, obs_d = obs_table.shape
    _, act_d = act_table.shape

    block_rows = 2000
    grid = n // block_rows

    return tuple(
        pl.pallas_call(
            _copy_body,
            grid=(grid,),
            in_specs=[
                pl.BlockSpec((block_rows, obs_d), lambda i: (i, 0)),
                pl.BlockSpec((block_rows, act_d), lambda i: (i, 0)),
            ],
            out_specs=[
                pl.BlockSpec((block_rows, obs_d), lambda i: (i, 0)),
                pl.BlockSpec((block_rows, act_d), lambda i: (i, 0)),
            ],
            out_shape=[
                jax.ShapeDtypeStruct(obs_table.shape, obs_table.dtype),
                jax.ShapeDtypeStruct(act_table.shape, act_table.dtype),
            ],
        )(obs_table, act_table)
    )


# speedup vs baseline: 17.7412x; 15.7585x over previous
import jax
import jax.numpy as jnp
from jax.experimental import pallas as pl

_N = 100000
_OBS_D = 17
_ACT_D = 6
_BC = 14336


def _copy_body(obs_ref, act_ref, obs_out, act_out):
    obs_out[...] = obs_ref[...]
    act_out[...] = act_ref[...]


def kernel(obs_table, act_table):
    obs_t = obs_table.T
    act_t = act_table.T

    grid = (_N + _BC - 1) // _BC
    obs_o, act_o = pl.pallas_call(
        _copy_body,
        grid=(grid,),
        in_specs=[
            pl.BlockSpec((_OBS_D, _BC), lambda i: (0, i)),
            pl.BlockSpec((_ACT_D, _BC), lambda i: (0, i)),
        ],
        out_specs=[
            pl.BlockSpec((_OBS_D, _BC), lambda i: (0, i)),
            pl.BlockSpec((_ACT_D, _BC), lambda i: (0, i)),
        ],
        out_shape=[
            jax.ShapeDtypeStruct((_OBS_D, _N), jnp.float32),
            jax.ShapeDtypeStruct((_ACT_D, _N), jnp.float32),
        ],
    )(obs_t, act_t)

    return (obs_o.T, act_o.T)


# speedup vs baseline: 19.2641x; 1.0858x over previous
import jax
import jax.numpy as jnp
from jax.experimental import pallas as pl

_N = 100000
_OBS_D = 17
_ACT_D = 6
_BC = 25088


def _copy_body(obs_ref, act_ref, obs_out, act_out):
    obs_out[...] = obs_ref[...]
    act_out[...] = act_ref[...]


def kernel(obs_table, act_table):
    obs_t = obs_table.T
    act_t = act_table.T

    grid = (_N + _BC - 1) // _BC
    obs_o, act_o = pl.pallas_call(
        _copy_body,
        grid=(grid,),
        in_specs=[
            pl.BlockSpec((_OBS_D, _BC), lambda i: (0, i)),
            pl.BlockSpec((_ACT_D, _BC), lambda i: (0, i)),
        ],
        out_specs=[
            pl.BlockSpec((_OBS_D, _BC), lambda i: (0, i)),
            pl.BlockSpec((_ACT_D, _BC), lambda i: (0, i)),
        ],
        out_shape=[
            jax.ShapeDtypeStruct((_OBS_D, _N), jnp.float32),
            jax.ShapeDtypeStruct((_ACT_D, _N), jnp.float32),
        ],
    )(obs_t, act_t)

    return (obs_o.T, act_o.T)


# speedup vs baseline: 20.4861x; 1.0634x over previous
import jax
import jax.numpy as jnp
from jax.experimental import pallas as pl

_N = 100000
_OBS_D = 17
_ACT_D = 6
_BC = 33408


def _copy_body(obs_ref, act_ref, obs_out, act_out):
    obs_out[...] = obs_ref[...]
    act_out[...] = act_ref[...]


def kernel(obs_table, act_table):
    obs_t = obs_table.T
    act_t = act_table.T

    grid = (_N + _BC - 1) // _BC
    obs_o, act_o = pl.pallas_call(
        _copy_body,
        grid=(grid,),
        in_specs=[
            pl.BlockSpec((_OBS_D, _BC), lambda i: (0, i)),
            pl.BlockSpec((_ACT_D, _BC), lambda i: (0, i)),
        ],
        out_specs=[
            pl.BlockSpec((_OBS_D, _BC), lambda i: (0, i)),
            pl.BlockSpec((_ACT_D, _BC), lambda i: (0, i)),
        ],
        out_shape=[
            jax.ShapeDtypeStruct((_OBS_D, _N), jnp.float32),
            jax.ShapeDtypeStruct((_ACT_D, _N), jnp.float32),
        ],
    )(obs_t, act_t)

    return (obs_o.T, act_o.T)


# speedup vs baseline: 22.3216x; 1.0896x over previous
import jax
import jax.numpy as jnp
from jax.experimental import pallas as pl

_N = 100000
_OBS_D = 17
_ACT_D = 6
_BC = 50048


def _copy_body(obs_ref, act_ref, obs_out, act_out):
    obs_out[...] = obs_ref[...]
    act_out[...] = act_ref[...]


def kernel(obs_table, act_table):
    obs_t = obs_table.T
    act_t = act_table.T

    grid = (_N + _BC - 1) // _BC
    obs_o, act_o = pl.pallas_call(
        _copy_body,
        grid=(grid,),
        in_specs=[
            pl.BlockSpec((_OBS_D, _BC), lambda i: (0, i)),
            pl.BlockSpec((_ACT_D, _BC), lambda i: (0, i)),
        ],
        out_specs=[
            pl.BlockSpec((_OBS_D, _BC), lambda i: (0, i)),
            pl.BlockSpec((_ACT_D, _BC), lambda i: (0, i)),
        ],
        out_shape=[
            jax.ShapeDtypeStruct((_OBS_D, _N), jnp.float32),
            jax.ShapeDtypeStruct((_ACT_D, _N), jnp.float32),
        ],
    )(obs_t, act_t)

    return (obs_o.T, act_o.T)
